# baseline (speedup 1.0000x reference)
import functools

import jax
import jax.numpy as jnp
from jax import lax
from jax.experimental import pallas as pl
from jax.experimental.pallas import tpu as pltpu
from jax.experimental.pallas import tpu_sc as plsc


def _proj_body(x_ref, w_ref, b_ref, o_ref):
    o_ref[...] = (
        jnp.dot(x_ref[...], w_ref[...], preferred_element_type=jnp.float32)
        + b_ref[0:1, :]
    )


def _project_table(word_table, W, b, pack=4, blk=2000):
    V, D = word_table.shape
    w_bd = jnp.kron(jnp.eye(pack, dtype=W.dtype), W.T)
    b_rep = jnp.broadcast_to(jnp.tile(b, pack)[None, :], (8, pack * D))
    x = word_table.reshape(V // pack, pack * D)
    n_rows = V // pack
    proj = pl.pallas_call(
        _proj_body,
        grid=(n_rows // blk,),
        in_specs=[
            pl.BlockSpec((blk, pack * D), lambda i: (i, 0)),
            pl.BlockSpec((pack * D, pack * D), lambda i: (0, 0)),
            pl.BlockSpec((8, pack * D), lambda i: (0, 0)),
        ],
        out_specs=pl.BlockSpec((blk, pack * D), lambda i: (i, 0)),
        out_shape=jax.ShapeDtypeStruct((n_rows, pack * D), jnp.float32),
    )(x, w_bd, b_rep)
    return proj.reshape(V, D)


_GL = 128


def _make_sc_gather(V, D, NW, per_w, ch):
    n_grp = per_w // _GL
    g_per_ch = ch // _GL
    n_pairs = per_w // (2 * ch)
    assert n_pairs * 2 * ch == per_w

    mesh = plsc.VectorSubcoreMesh(core_axis_name="c", subcore_axis_name="s")

    @functools.partial(
        pl.kernel,
        out_type=jax.ShapeDtypeStruct((NW * per_w, D), jnp.float32),
        mesh=mesh,
        scratch_types=[
            pltpu.VMEM((n_grp, _GL), jnp.int32),
            pltpu.VMEM((2, ch, D), jnp.float32),
            pltpu.SemaphoreType.DMA,
            pltpu.SemaphoreType.DMA,
        ],
        compiler_params=pltpu.CompilerParams(use_tc_tiling_on_sc=False),
    )
    def sc_gather(ptab_hbm, idx_hbm, out_hbm, idx_v, rows_v, sem0, sem1):
        wid = lax.axis_index("s") * 2 + lax.axis_index("c")
        base = wid * per_w
        pltpu.sync_copy(idx_hbm.at[wid], idx_v)
        sems = (sem0, sem1)

        @pl.loop(0, n_pairs)
        def _pair(i):
            handles = ([], [])
            for bb in range(2):
                cidx = i * 2 + bb
                for g in range(g_per_ch):
                    row = cidx * g_per_ch + g
                    handles[bb].append(
                        pltpu.async_copy(
                            ptab_hbm.at[idx_v.at[row]],
                            rows_v.at[bb, pl.ds(g * _GL, _GL)],
                            sems[bb],
                        )
                    )
            for bb in range(2):
                for h in handles[bb]:
                    h.wait()
                cidx = i * 2 + bb
                pltpu.sync_copy(
                    rows_v.at[bb], out_hbm.at[pl.ds(base + cidx * ch, ch)]
                )

    return sc_gather


def kernel(inputs, word_table, W, b):
    V, D = word_table.shape
    B, L = inputs.shape
    ptab = _project_table(word_table, W, b)

    NW = 32
    total = B * L
    per_w = total // NW
    idx3 = inputs.reshape(NW, per_w // _GL, _GL)
    out2 = _make_sc_gather(V, D, NW, per_w, ch=512)(ptab, idx3)
    return out2.reshape(B, L, D)
